# baseline (speedup 1.0000x reference)
import jax
import jax.numpy as jnp
from jax.experimental import pallas as pl
from jax.experimental.pallas import tpu as pltpu

_N = 2048
_F = 512
_H = 8
_E = 64
_VW = 128
_BLK = 256
_NB = _N // _BLK


def _gat_kernel(x_ref, w_ref, ws_ref, adj_ref, bias_ref, out_ref,
                vaug_s, ss_s, snT_s):
    i = pl.program_id(0)

    @pl.when(i < _NB)
    def _phase_features():
        feat = jnp.dot(x_ref[...], w_ref[...], preferred_element_type=jnp.float32)
        s = jnp.dot(feat, ws_ref[...], preferred_element_type=jnp.float32)
        ss_s[pl.ds(i * _BLK, _BLK), :] = s[:, :_H]
        snT_s[:, pl.ds(i * _BLK, _BLK)] = s[:, _H:].T
        ones_pad = jnp.where(
            jax.lax.broadcasted_iota(jnp.int32, (_BLK, _VW - _E), 1) == 0, 1.0, 0.0
        ).astype(jnp.bfloat16)
        feat16 = feat.astype(jnp.bfloat16)
        for h in range(_H):
            vaug_s[pl.ds(i * _BLK, _BLK), h * _VW : h * _VW + _E] = (
                feat16[:, h * _E : (h + 1) * _E])
            vaug_s[pl.ds(i * _BLK, _BLK), h * _VW + _E : (h + 1) * _VW] = ones_pad

    @pl.when(i >= _NB)
    def _phase_attention():
        j = i - _NB
        adj = adj_ref[...].astype(jnp.bfloat16)
        sself = ss_s[pl.ds(j * _BLK, _BLK), :]
        for h in range(_H):
            a_i = sself[:, h : h + 1]
            a_j = snT_s[h : h + 1, :]
            r = jnp.exp(-0.8 * a_i).astype(jnp.bfloat16)
            q = jnp.exp(a_j).astype(jnp.bfloat16)
            q2 = jnp.exp(0.2 * a_j).astype(jnp.bfloat16)
            e = adj * jnp.maximum(q, r * q2)
            o2 = jnp.dot(
                e, vaug_s[:, h * _VW : (h + 1) * _VW],
                preferred_element_type=jnp.float32,
            )
            o = (o2[:, :_E] / o2[:, _E : _E + 1]
                 + bias_ref[:, h * _E : (h + 1) * _E])
            out_ref[:, h * _E : (h + 1) * _E] = jnp.maximum(o, 0.0)


def kernel(x, adj, weight, att_self_weight, att_neighs_weight, bias_weight):
    eye = jnp.eye(_H, dtype=jnp.float32)
    sel = jnp.repeat(eye, _E, axis=0)
    ws = jnp.concatenate(
        [sel * att_self_weight.reshape(_H * _E, 1),
         sel * att_neighs_weight.reshape(_H * _E, 1)], axis=1)
    bias2d = bias_weight.reshape(1, _H * _E)

    out = pl.pallas_call(
        _gat_kernel,
        grid=(2 * _NB,),
        in_specs=[
            pl.BlockSpec((_BLK, _F), lambda i: (jnp.minimum(i, _NB - 1), 0)),
            pl.BlockSpec((_F, _H * _E), lambda i: (0, 0)),
            pl.BlockSpec((_H * _E, 2 * _H), lambda i: (0, 0)),
            pl.BlockSpec((_BLK, _N), lambda i: (jnp.maximum(i - _NB, 0), 0)),
            pl.BlockSpec((1, _H * _E), lambda i: (0, 0)),
        ],
        out_specs=pl.BlockSpec((_BLK, _H * _E), lambda i: (jnp.maximum(i - _NB, 0), 0)),
        out_shape=jax.ShapeDtypeStruct((_N, _H * _E), jnp.float32),
        scratch_shapes=[
            pltpu.VMEM((_N, _H * _VW), jnp.bfloat16),
            pltpu.VMEM((_N, _H), jnp.float32),
            pltpu.VMEM((_H, _N), jnp.float32),
        ],
    )(x, weight, ws, adj, bias2d)

    return out


# speedup vs baseline: 1.2161x; 1.2161x over previous
import jax
import jax.numpy as jnp
from jax.experimental import pallas as pl
from jax.experimental.pallas import tpu as pltpu

_N = 2048
_F = 512
_H = 8
_E = 64
_VW = 128
_BLK = 1024
_NB = _N // _BLK


def _gat_kernel(x_ref, w_ref, ws_ref, adj_ref, bias_ref, out_ref,
                vaug_s, ss_s, snT_s):
    i = pl.program_id(0)

    @pl.when(i < _NB)
    def _phase_features():
        feat = jnp.dot(x_ref[...], w_ref[...], preferred_element_type=jnp.float32)
        s = jnp.dot(feat, ws_ref[...], preferred_element_type=jnp.float32)
        ss_s[pl.ds(i * _BLK, _BLK), :] = s[:, :_H]
        snT_s[:, pl.ds(i * _BLK, _BLK)] = s[:, _H:].T
        ones_pad = jnp.where(
            jax.lax.broadcasted_iota(jnp.int32, (_BLK, _VW - _E), 1) == 0, 1.0, 0.0
        ).astype(jnp.bfloat16)
        feat16 = feat.astype(jnp.bfloat16)
        for h in range(_H):
            vaug_s[pl.ds(i * _BLK, _BLK), h * _VW : h * _VW + _E] = (
                feat16[:, h * _E : (h + 1) * _E])
            vaug_s[pl.ds(i * _BLK, _BLK), h * _VW + _E : (h + 1) * _VW] = ones_pad

    @pl.when(i >= _NB)
    def _phase_attention():
        j = i - _NB
        adj = adj_ref[...].astype(jnp.bfloat16)
        sself = ss_s[pl.ds(j * _BLK, _BLK), :]
        for h in range(_H):
            a_i = sself[:, h : h + 1]
            a_j = snT_s[h : h + 1, :]
            r = jnp.exp(-0.8 * a_i).astype(jnp.bfloat16)
            q = jnp.exp(a_j).astype(jnp.bfloat16)
            q2 = jnp.exp(0.2 * a_j).astype(jnp.bfloat16)
            e = adj * jnp.maximum(q, r * q2)
            o2 = jnp.dot(
                e, vaug_s[:, h * _VW : (h + 1) * _VW],
                preferred_element_type=jnp.float32,
            )
            o = (o2[:, :_E] / o2[:, _E : _E + 1]
                 + bias_ref[:, h * _E : (h + 1) * _E])
            out_ref[:, h * _E : (h + 1) * _E] = jnp.maximum(o, 0.0)


def kernel(x, adj, weight, att_self_weight, att_neighs_weight, bias_weight):
    eye = jnp.eye(_H, dtype=jnp.float32)
    sel = jnp.repeat(eye, _E, axis=0)
    ws = jnp.concatenate(
        [sel * att_self_weight.reshape(_H * _E, 1),
         sel * att_neighs_weight.reshape(_H * _E, 1)], axis=1)
    bias2d = bias_weight.reshape(1, _H * _E)

    out = pl.pallas_call(
        _gat_kernel,
        grid=(2 * _NB,),
        in_specs=[
            pl.BlockSpec((_BLK, _F), lambda i: (jnp.minimum(i, _NB - 1), 0)),
            pl.BlockSpec((_F, _H * _E), lambda i: (0, 0)),
            pl.BlockSpec((_H * _E, 2 * _H), lambda i: (0, 0)),
            pl.BlockSpec((_BLK, _N), lambda i: (jnp.maximum(i - _NB, 0), 0)),
            pl.BlockSpec((1, _H * _E), lambda i: (0, 0)),
        ],
        out_specs=pl.BlockSpec((_BLK, _H * _E), lambda i: (jnp.maximum(i - _NB, 0), 0)),
        out_shape=jax.ShapeDtypeStruct((_N, _H * _E), jnp.float32),
        scratch_shapes=[
            pltpu.VMEM((_N, _H * _VW), jnp.bfloat16),
            pltpu.VMEM((_N, _H), jnp.float32),
            pltpu.VMEM((_H, _N), jnp.float32),
        ],
    )(x, weight, ws, adj, bias2d)

    return out


# speedup vs baseline: 1.2288x; 1.0105x over previous
import jax
import jax.numpy as jnp
from jax.experimental import pallas as pl
from jax.experimental.pallas import tpu as pltpu

_N = 2048
_F = 512
_H = 8
_E = 64
_VW = 128
_BLK = 512
_NB = _N // _BLK


def _gat_kernel(x_ref, w_ref, ws_ref, adj_ref, bias_ref, out_ref,
                vaug_s, ss_s, snT_s):
    i = pl.program_id(0)

    @pl.when(i < _NB)
    def _phase_features():
        feat = jnp.dot(x_ref[...], w_ref[...], preferred_element_type=jnp.float32)
        s = jnp.dot(feat, ws_ref[...], preferred_element_type=jnp.float32)
        ss_s[pl.ds(i * _BLK, _BLK), :] = s[:, :_H]
        snT_s[:, pl.ds(i * _BLK, _BLK)] = s[:, _H:].T
        ones_pad = jnp.where(
            jax.lax.broadcasted_iota(jnp.int32, (_BLK, _VW - _E), 1) == 0, 1.0, 0.0
        ).astype(jnp.bfloat16)
        feat16 = feat.astype(jnp.bfloat16)
        for h in range(_H):
            vaug_s[pl.ds(i * _BLK, _BLK), h * _VW : h * _VW + _E] = (
                feat16[:, h * _E : (h + 1) * _E])
            vaug_s[pl.ds(i * _BLK, _BLK), h * _VW + _E : (h + 1) * _VW] = ones_pad

    @pl.when(i >= _NB)
    def _phase_attention():
        j = i - _NB
        adj = adj_ref[...].astype(jnp.bfloat16)
        sself = ss_s[pl.ds(j * _BLK, _BLK), :]
        for h in range(_H):
            a_i = sself[:, h : h + 1]
            a_j = snT_s[h : h + 1, :]
            r = jnp.exp(-0.8 * a_i).astype(jnp.bfloat16)
            q = jnp.exp(a_j).astype(jnp.bfloat16)
            q2 = jnp.exp(0.2 * a_j).astype(jnp.bfloat16)
            e = adj * jnp.maximum(q, r * q2)
            o2 = jnp.dot(
                e, vaug_s[:, h * _VW : (h + 1) * _VW],
                preferred_element_type=jnp.float32,
            )
            o = (o2[:, :_E] / o2[:, _E : _E + 1]
                 + bias_ref[:, h * _E : (h + 1) * _E])
            out_ref[:, h * _E : (h + 1) * _E] = jnp.maximum(o, 0.0)


def kernel(x, adj, weight, att_self_weight, att_neighs_weight, bias_weight):
    eye = jnp.eye(_H, dtype=jnp.float32)
    sel = jnp.repeat(eye, _E, axis=0)
    ws = jnp.concatenate(
        [sel * att_self_weight.reshape(_H * _E, 1),
         sel * att_neighs_weight.reshape(_H * _E, 1)], axis=1)
    bias2d = bias_weight.reshape(1, _H * _E)

    out = pl.pallas_call(
        _gat_kernel,
        grid=(2 * _NB,),
        in_specs=[
            pl.BlockSpec((_BLK, _F), lambda i: (jnp.minimum(i, _NB - 1), 0)),
            pl.BlockSpec((_F, _H * _E), lambda i: (0, 0)),
            pl.BlockSpec((_H * _E, 2 * _H), lambda i: (0, 0)),
            pl.BlockSpec((_BLK, _N), lambda i: (jnp.maximum(i - _NB, 0), 0)),
            pl.BlockSpec((1, _H * _E), lambda i: (0, 0)),
        ],
        out_specs=pl.BlockSpec((_BLK, _H * _E), lambda i: (jnp.maximum(i - _NB, 0), 0)),
        out_shape=jax.ShapeDtypeStruct((_N, _H * _E), jnp.float32),
        scratch_shapes=[
            pltpu.VMEM((_N, _H * _VW), jnp.bfloat16),
            pltpu.VMEM((_N, _H), jnp.float32),
            pltpu.VMEM((_H, _N), jnp.float32),
        ],
    )(x, weight, ws, adj, bias2d)

    return out
